# baseline (speedup 1.0000x reference)
import jax
import jax.numpy as jnp
import numpy as np
from jax import lax
from jax.experimental import pallas as pl
from jax.experimental.pallas import tpu as pltpu

H = 1024
W = 1024
LATENT_C = 4
HID = 64
TWO_PI = 2.0 * np.pi
B = 1024


def _body(crow_ref, srow_ref, rp_ref, ct_ref,
          tab_ref, Wtrig_ref, ba_ref, W0lat_ref, Wf1_ref, Wf2_ref,
          bgam_ref, bbet_ref, W1_ref, b1_ref, W2_ref, b2_ref,
          out_ref, scr):
    masks = [lax.broadcasted_iota(jnp.int32, (8, 128), 0) == u
             for u in range(1, 8)]

    def gstep(k, carry):
        base = k * 8
        acc = None
        for u in range(8):
            r = pl.multiple_of(crow_ref[0, 0, base + u], 8)
            s = srow_ref[0, 0, base + u]
            chunk = tab_ref[pl.ds(r, 8), :]
            rolled = pltpu.roll(chunk, u - s, axis=0)
            acc = rolled if u == 0 else jnp.where(masks[u - 1], rolled, acc)
        scr[pl.ds(pl.multiple_of(base, 8), 8), :] = acc
        return carry

    lax.fori_loop(0, B // 8, gstep, 0)

    rp = rp_ref[0]
    ct = ct_ref[0]
    xcol = rp[:, 0:1]
    lane = lax.broadcasted_iota(jnp.int32, (B, 128), 1)
    lidx = (xcol & 31) * 4 + (lane & 3)
    slabs = scr[...]
    g = jnp.take_along_axis(slabs, lidx, axis=1)[:, 0:4]

    rpT = jnp.transpose(rp)
    ctT = jnp.transpose(ct)
    xf = rpT[0:1, :].astype(jnp.float32) * (1.0 / W)
    yf = rpT[1:2, :].astype(jnp.float32) * (1.0 / H)
    c0 = ctT[0:1, :]
    c1 = ctT[1:2, :]

    def six(sa, ca):
        s2 = 2.0 * sa * ca
        c2 = 1.0 - 2.0 * sa * sa
        s3 = sa * c2 + ca * s2
        c3 = ca * c2 - sa * s2
        return s2, c2, s3, c3

    x2 = xf * TWO_PI
    y2 = yf * TWO_PI
    t2 = c0 * TWO_PI
    sx, cx = jnp.sin(x2), jnp.cos(x2)
    sy, cy = jnp.sin(y2), jnp.cos(y2)
    st, ct_ = jnp.sin(t2), jnp.cos(t2)
    s2x, c2x, s3x, c3x = six(sx, cx)
    s2y, c2y, s3y, c3y = six(sy, cy)
    s2t, c2t, s3t, c3t = six(st, ct_)

    zr = jnp.zeros_like(xf)
    stack = jnp.concatenate(
        [xf, yf, sx, cx, s2x, c2x, s3x, c3x,
         sy, cy, s2y, c2y, s3y, c3y,
         c0, c1, st, ct_, s2t, c2t, s3t, c3t, zr, zr], axis=0)
    trig_t = jnp.transpose(stack)

    u1 = jnp.dot(trig_t, Wtrig_ref[...],
                 preferred_element_type=jnp.float32) + ba_ref[...]
    h0 = u1[:, 0:64] + jnp.dot(g, W0lat_ref[...],
                               preferred_element_type=jnp.float32)
    film = jnp.maximum(u1[:, 64:112], 0.0)

    gamma = jnp.dot(film, Wf1_ref[...],
                    preferred_element_type=jnp.float32) + bgam_ref[...]
    beta = jnp.dot(film, Wf2_ref[...],
                   preferred_element_type=jnp.float32) + bbet_ref[...]

    def gelu(z):
        return 0.5 * z * (1.0 + lax.erf(z * np.float32(0.7071067811865476)))

    h = gelu(gamma * h0 + beta)
    h2 = gelu(jnp.dot(h, W1_ref[...],
                      preferred_element_type=jnp.float32) + b1_ref[...])
    o = jnp.dot(h2, W2_ref[...],
                preferred_element_type=jnp.float32) + b2_ref[...]
    out_ref[...] = jax.nn.sigmoid(o)


@jax.jit
def kernel(raw_pos, control, latent, Wc, bc, Wt, bt, Wp, bp, Wf, bf,
           W0, b0, W1, b1, W2, b2):
    N = raw_pos.shape[0]
    NB = N // B
    f32 = jnp.float32

    rp = raw_pos.astype(jnp.int32).reshape(NB, B, 2)
    ct = control.reshape(NB, B, 2)
    p = (raw_pos[:, 1].astype(jnp.int32) * W
         + raw_pos[:, 0].astype(jnp.int32))
    crow = ((p >> 8) << 3).reshape(NB, 1, B)
    srow = ((p >> 5) & 7).reshape(NB, 1, B)
    tab = latent.reshape(H * W * LATENT_C // 128, 128)

    Wtrig = jnp.zeros((24, 112), f32)
    Wtrig = Wtrig.at[0, 0:64].set(W0[4])
    Wtrig = Wtrig.at[1, 0:64].set(W0[5])
    Wtrig = Wtrig.at[2, 0:64].set(W0[6])
    Wtrig = Wtrig.at[8, 0:64].set(W0[7])
    Wtrig = Wtrig.at[4, 0:64].set(W0[8])
    Wtrig = Wtrig.at[10, 0:64].set(W0[9])
    Wtrig = Wtrig.at[14, 64:80].set(Wc[0])
    Wtrig = Wtrig.at[15, 64:80].set(Wc[1])
    Wtrig = Wtrig.at[2:14, 80:96].set(Wp)
    Wtrig = Wtrig.at[16:22, 96:112].set(Wt)
    ba = jnp.concatenate([b0, bc, bp, bt]).reshape(1, 112)
    W0lat = W0[0:4]
    Wf1 = Wf[:, 0:64]
    Wf2 = Wf[:, 64:128]
    bgam = bf[0:64].reshape(1, 64)
    bbet = bf[64:128].reshape(1, 64)
    b1m = b1.reshape(1, 64)
    b2m = b2.reshape(1, 4)

    full = lambda shp: pl.BlockSpec(shp, lambda i: tuple(0 for _ in shp))
    grid_spec = pltpu.PrefetchScalarGridSpec(
        num_scalar_prefetch=0,
        grid=(NB,),
        in_specs=[
            pl.BlockSpec((1, 1, B), lambda i: (i, 0, 0),
                         memory_space=pltpu.SMEM),
            pl.BlockSpec((1, 1, B), lambda i: (i, 0, 0),
                         memory_space=pltpu.SMEM),
            pl.BlockSpec((1, B, 2), lambda i: (i, 0, 0)),
            pl.BlockSpec((1, B, 2), lambda i: (i, 0, 0)),
            full((H * W * LATENT_C // 128, 128)),
            full((24, 112)),
            full((1, 112)),
            full((4, 64)),
            full((48, 64)),
            full((48, 64)),
            full((1, 64)),
            full((1, 64)),
            full((64, 64)),
            full((1, 64)),
            full((64, 4)),
            full((1, 4)),
        ],
        out_specs=pl.BlockSpec((B, 4), lambda i: (i, 0)),
        scratch_shapes=[pltpu.VMEM((B, 128), f32)],
    )
    out = pl.pallas_call(
        _body,
        grid_spec=grid_spec,
        out_shape=jax.ShapeDtypeStruct((N, 4), f32),
        compiler_params=pltpu.CompilerParams(
            dimension_semantics=("arbitrary",),
        ),
    )(crow, srow, rp, ct, tab, Wtrig, ba, W0lat, Wf1, Wf2,
      bgam, bbet, W1, b1m, W2, b2m)
    return out


# speedup vs baseline: 1.0009x; 1.0009x over previous
import jax
import jax.numpy as jnp
import numpy as np
from jax import lax
from jax.experimental import pallas as pl
from jax.experimental.pallas import tpu as pltpu

H = 1024
W = 1024
LATENT_C = 4
HID = 64
TWO_PI = 2.0 * np.pi
B = 1024


def _body(crow_ref, srow_ref, rp_ref, ct_ref,
          tab_ref, Wtrig_ref, ba_ref, W0lat_ref, Wf1_ref, Wf2_ref,
          bgam_ref, bbet_ref, W1_ref, b1_ref, W2_ref, b2_ref,
          out_ref, scr):
    masks = [lax.broadcasted_iota(jnp.int32, (8, 128), 0) == u
             for u in range(1, 8)]

    def gstep(k, carry):
        base = k * 8
        acc = None
        for u in range(8):
            r = pl.multiple_of(crow_ref[0, 0, base + u], 8)
            s = srow_ref[0, 0, base + u]
            chunk = tab_ref[pl.ds(r, 8), :]
            rolled = pltpu.roll(chunk, u - s, axis=0)
            acc = rolled if u == 0 else jnp.where(masks[u - 1], rolled, acc)
        scr[pl.ds(pl.multiple_of(base, 8), 8), :] = acc
        return carry

    lax.fori_loop(0, B // 8, gstep, 0)

    rp = rp_ref[0]
    ct = ct_ref[0]
    xcol = rp[:, 0:1]
    lane = lax.broadcasted_iota(jnp.int32, (B, 128), 1)
    lidx = (xcol & 31) * 4 + (lane & 3)
    slabs = scr[...]
    g = jnp.take_along_axis(slabs, lidx, axis=1)[:, 0:4]

    rpT = jnp.transpose(rp)
    ctT = jnp.transpose(ct)
    xf = rpT[0:1, :].astype(jnp.float32) * (1.0 / W)
    yf = rpT[1:2, :].astype(jnp.float32) * (1.0 / H)
    c0 = ctT[0:1, :]
    c1 = ctT[1:2, :]

    def six(sa, ca):
        s2 = 2.0 * sa * ca
        c2 = 1.0 - 2.0 * sa * sa
        s3 = sa * c2 + ca * s2
        c3 = ca * c2 - sa * s2
        return s2, c2, s3, c3

    x2 = xf * TWO_PI
    y2 = yf * TWO_PI
    t2 = c0 * TWO_PI
    sx, cx = jnp.sin(x2), jnp.cos(x2)
    sy, cy = jnp.sin(y2), jnp.cos(y2)
    st, ct_ = jnp.sin(t2), jnp.cos(t2)
    s2x, c2x, s3x, c3x = six(sx, cx)
    s2y, c2y, s3y, c3y = six(sy, cy)
    s2t, c2t, s3t, c3t = six(st, ct_)

    zr = jnp.zeros_like(xf)
    stack = jnp.concatenate(
        [xf, yf, sx, cx, s2x, c2x, s3x, c3x,
         sy, cy, s2y, c2y, s3y, c3y,
         c0, c1, st, ct_, s2t, c2t, s3t, c3t, zr, zr], axis=0)
    trig_t = jnp.transpose(stack)

    u1 = jnp.dot(trig_t, Wtrig_ref[...],
                 preferred_element_type=jnp.float32) + ba_ref[...]
    h0 = u1[:, 0:64] + jnp.dot(g, W0lat_ref[...],
                               preferred_element_type=jnp.float32)
    film = jnp.maximum(u1[:, 64:112], 0.0)

    gamma = jnp.dot(film, Wf1_ref[...],
                    preferred_element_type=jnp.float32) + bgam_ref[...]
    beta = jnp.dot(film, Wf2_ref[...],
                   preferred_element_type=jnp.float32) + bbet_ref[...]

    def gelu(z):
        return 0.5 * z * (1.0 + lax.erf(z * np.float32(0.7071067811865476)))

    h = gelu(gamma * h0 + beta)
    h2 = gelu(jnp.dot(h, W1_ref[...],
                      preferred_element_type=jnp.float32) + b1_ref[...])
    o = jnp.dot(h2, W2_ref[...],
                preferred_element_type=jnp.float32) + b2_ref[...]
    out_ref[...] = jax.nn.sigmoid(o)


@jax.jit
def kernel(raw_pos, control, latent, Wc, bc, Wt, bt, Wp, bp, Wf, bf,
           W0, b0, W1, b1, W2, b2):
    N = raw_pos.shape[0]
    NB = N // B
    f32 = jnp.float32

    rp = raw_pos.astype(jnp.int32).reshape(NB, B, 2)
    ct = control.reshape(NB, B, 2)
    p = (raw_pos[:, 1].astype(jnp.int32) * W
         + raw_pos[:, 0].astype(jnp.int32))
    crow = ((p >> 8) << 3).reshape(NB, 1, B)
    srow = ((p >> 5) & 7).reshape(NB, 1, B)
    tab = latent.reshape(H * W * LATENT_C // 128, 128)

    Wtrig = jnp.zeros((24, 112), f32)
    Wtrig = Wtrig.at[0, 0:64].set(W0[4])
    Wtrig = Wtrig.at[1, 0:64].set(W0[5])
    Wtrig = Wtrig.at[2, 0:64].set(W0[6])
    Wtrig = Wtrig.at[8, 0:64].set(W0[7])
    Wtrig = Wtrig.at[4, 0:64].set(W0[8])
    Wtrig = Wtrig.at[10, 0:64].set(W0[9])
    Wtrig = Wtrig.at[14, 64:80].set(Wc[0])
    Wtrig = Wtrig.at[15, 64:80].set(Wc[1])
    Wtrig = Wtrig.at[2:14, 80:96].set(Wp)
    Wtrig = Wtrig.at[16:22, 96:112].set(Wt)
    ba = jnp.concatenate([b0, bc, bp, bt]).reshape(1, 112)
    W0lat = W0[0:4]
    Wf1 = Wf[:, 0:64]
    Wf2 = Wf[:, 64:128]
    bgam = bf[0:64].reshape(1, 64)
    bbet = bf[64:128].reshape(1, 64)
    b1m = b1.reshape(1, 64)
    b2m = b2.reshape(1, 4)

    full = lambda shp: pl.BlockSpec(shp, lambda i: tuple(0 for _ in shp))
    grid_spec = pltpu.PrefetchScalarGridSpec(
        num_scalar_prefetch=0,
        grid=(NB,),
        in_specs=[
            pl.BlockSpec((1, 1, B), lambda i: (i, 0, 0),
                         memory_space=pltpu.SMEM),
            pl.BlockSpec((1, 1, B), lambda i: (i, 0, 0),
                         memory_space=pltpu.SMEM),
            pl.BlockSpec((1, B, 2), lambda i: (i, 0, 0)),
            pl.BlockSpec((1, B, 2), lambda i: (i, 0, 0)),
            full((H * W * LATENT_C // 128, 128)),
            full((24, 112)),
            full((1, 112)),
            full((4, 64)),
            full((48, 64)),
            full((48, 64)),
            full((1, 64)),
            full((1, 64)),
            full((64, 64)),
            full((1, 64)),
            full((64, 4)),
            full((1, 4)),
        ],
        out_specs=pl.BlockSpec((B, 4), lambda i: (i, 0)),
        scratch_shapes=[pltpu.VMEM((B, 128), f32)],
    )
    out = pl.pallas_call(
        _body,
        grid_spec=grid_spec,
        out_shape=jax.ShapeDtypeStruct((N, 4), f32),
        compiler_params=pltpu.CompilerParams(
            dimension_semantics=("parallel",),
        ),
    )(crow, srow, rp, ct, tab, Wtrig, ba, W0lat, Wf1, Wf2,
      bgam, bbet, W1, b1m, W2, b2m)
    return out


# speedup vs baseline: 1.4394x; 1.4381x over previous
import jax
import jax.numpy as jnp
import numpy as np
from jax import lax
from jax.experimental import pallas as pl
from jax.experimental.pallas import tpu as pltpu

H = 1024
W = 1024
LATENT_C = 4
HID = 64
TWO_PI = 2.0 * np.pi
B = 1024


def _body(crow_ref, sh_ref, pcol_ref, xi_ref, yi_ref, c0_ref, c1_ref,
          tab_ref, Wtrig_ref, ba_ref, W0lat_ref, Wf1_ref, Wf2_ref,
          bgam_ref, bbet_ref, W1_ref, b1_ref, W2_ref, b2_ref,
          out_ref, scr):
    def gstep(k, carry):
        base = pl.multiple_of(k * 8, 8)
        for u in range(8):
            r = pl.multiple_of(crow_ref[0, 0, base + u], 8)
            s = sh_ref[0, 0, base + u]
            chunk = tab_ref[pl.ds(r, 8), :]
            rolled = pltpu.roll(chunk, s, axis=0)
            scr[pl.ds(base + u, 1), :] = rolled[u:u + 1, :]
        return carry

    lax.fori_loop(0, B // 8, gstep, 0)

    pcol = pcol_ref[0]
    lane = lax.broadcasted_iota(jnp.int32, (B, 128), 1)
    lidx = (pcol & 31) * 4 + (lane & 3)
    g = jnp.take_along_axis(scr[...], lidx, axis=1)[:, 0:4]

    xf = xi_ref[0].astype(jnp.float32) * (1.0 / W)
    yf = yi_ref[0].astype(jnp.float32) * (1.0 / H)
    c0 = c0_ref[0]
    c1 = c1_ref[0]

    def six(sa, ca):
        s2 = 2.0 * sa * ca
        c2 = 1.0 - 2.0 * sa * sa
        s3 = sa * c2 + ca * s2
        c3 = ca * c2 - sa * s2
        return s2, c2, s3, c3

    x2 = xf * TWO_PI
    y2 = yf * TWO_PI
    t2 = c0 * TWO_PI
    sx, cx = jnp.sin(x2), jnp.cos(x2)
    sy, cy = jnp.sin(y2), jnp.cos(y2)
    st, ct = jnp.sin(t2), jnp.cos(t2)
    s2x, c2x, s3x, c3x = six(sx, cx)
    s2y, c2y, s3y, c3y = six(sy, cy)
    s2t, c2t, s3t, c3t = six(st, ct)

    zr = jnp.zeros_like(xf)
    stack = jnp.concatenate(
        [xf, yf, sx, cx, s2x, c2x, s3x, c3x,
         sy, cy, s2y, c2y, s3y, c3y,
         c0, c1, st, ct, s2t, c2t, s3t, c3t, zr, zr], axis=0)
    trig_t = jnp.transpose(stack)

    u1 = jnp.dot(trig_t, Wtrig_ref[...],
                 preferred_element_type=jnp.float32) + ba_ref[...]
    h0 = u1[:, 0:64] + jnp.dot(g, W0lat_ref[...],
                               preferred_element_type=jnp.float32)
    film = jnp.maximum(u1[:, 64:112], 0.0)

    gamma = jnp.dot(film, Wf1_ref[...],
                    preferred_element_type=jnp.float32) + bgam_ref[...]
    beta = jnp.dot(film, Wf2_ref[...],
                   preferred_element_type=jnp.float32) + bbet_ref[...]

    def gelu(z):
        return 0.5 * z * (1.0 + lax.erf(z * np.float32(0.7071067811865476)))

    h = gelu(gamma * h0 + beta)
    h2 = gelu(jnp.dot(h, W1_ref[...],
                      preferred_element_type=jnp.float32) + b1_ref[...])
    o = jnp.dot(h2, W2_ref[...],
                preferred_element_type=jnp.float32) + b2_ref[...]
    out_ref[...] = jax.nn.sigmoid(o)


@jax.jit
def kernel(raw_pos, control, latent, Wc, bc, Wt, bt, Wp, bp, Wf, bf,
           W0, b0, W1, b1, W2, b2):
    N = raw_pos.shape[0]
    NB = N // B
    f32 = jnp.float32

    x = raw_pos[:, 0].astype(jnp.int32)
    y = raw_pos[:, 1].astype(jnp.int32)
    p = y * W + x
    iota = jnp.arange(N, dtype=jnp.int32)
    crow = ((p >> 8) << 3).reshape(NB, 1, B)
    sh = (((iota & 7) - ((p >> 5) & 7)) & 7).reshape(NB, 1, B)
    pcol = p.reshape(NB, B, 1)
    xi = x.reshape(NB, 1, B)
    yi = y.reshape(NB, 1, B)
    c0 = control[:, 0].reshape(NB, 1, B)
    c1 = control[:, 1].reshape(NB, 1, B)
    tab = latent.reshape(H * W * LATENT_C // 128, 128)

    Wtrig = jnp.zeros((24, 112), f32)
    Wtrig = Wtrig.at[0, 0:64].set(W0[4])
    Wtrig = Wtrig.at[1, 0:64].set(W0[5])
    Wtrig = Wtrig.at[2, 0:64].set(W0[6])
    Wtrig = Wtrig.at[8, 0:64].set(W0[7])
    Wtrig = Wtrig.at[4, 0:64].set(W0[8])
    Wtrig = Wtrig.at[10, 0:64].set(W0[9])
    Wtrig = Wtrig.at[14, 64:80].set(Wc[0])
    Wtrig = Wtrig.at[15, 64:80].set(Wc[1])
    Wtrig = Wtrig.at[2:14, 80:96].set(Wp)
    Wtrig = Wtrig.at[16:22, 96:112].set(Wt)
    ba = jnp.concatenate([b0, bc, bp, bt]).reshape(1, 112)
    W0lat = W0[0:4]
    Wf1 = Wf[:, 0:64]
    Wf2 = Wf[:, 64:128]
    bgam = bf[0:64].reshape(1, 64)
    bbet = bf[64:128].reshape(1, 64)
    b1m = b1.reshape(1, 64)
    b2m = b2.reshape(1, 4)

    full = lambda shp: pl.BlockSpec(shp, lambda i: tuple(0 for _ in shp))
    grid_spec = pltpu.PrefetchScalarGridSpec(
        num_scalar_prefetch=0,
        grid=(NB,),
        in_specs=[
            pl.BlockSpec((1, 1, B), lambda i: (i, 0, 0),
                         memory_space=pltpu.SMEM),
            pl.BlockSpec((1, 1, B), lambda i: (i, 0, 0),
                         memory_space=pltpu.SMEM),
            pl.BlockSpec((1, B, 1), lambda i: (i, 0, 0)),
            pl.BlockSpec((1, 1, B), lambda i: (i, 0, 0)),
            pl.BlockSpec((1, 1, B), lambda i: (i, 0, 0)),
            pl.BlockSpec((1, 1, B), lambda i: (i, 0, 0)),
            pl.BlockSpec((1, 1, B), lambda i: (i, 0, 0)),
            full((H * W * LATENT_C // 128, 128)),
            full((24, 112)),
            full((1, 112)),
            full((4, 64)),
            full((48, 64)),
            full((48, 64)),
            full((1, 64)),
            full((1, 64)),
            full((64, 64)),
            full((1, 64)),
            full((64, 4)),
            full((1, 4)),
        ],
        out_specs=pl.BlockSpec((B, 4), lambda i: (i, 0)),
        scratch_shapes=[pltpu.VMEM((B, 128), f32)],
    )
    out = pl.pallas_call(
        _body,
        grid_spec=grid_spec,
        out_shape=jax.ShapeDtypeStruct((N, 4), f32),
        compiler_params=pltpu.CompilerParams(
            dimension_semantics=("parallel",),
        ),
    )(crow, sh, pcol, xi, yi, c0, c1, tab, Wtrig, ba, W0lat, Wf1, Wf2,
      bgam, bbet, W1, b1m, W2, b2m)
    return out


# speedup vs baseline: 1.4412x; 1.0013x over previous
import jax
import jax.numpy as jnp
import numpy as np
from jax import lax
from jax.experimental import pallas as pl
from jax.experimental.pallas import tpu as pltpu

H = 1024
W = 1024
LATENT_C = 4
HID = 64
TWO_PI = 2.0 * np.pi
B = 1024


def _body(crow_ref, sh_ref, pcol_ref, prow_ref, c0_ref, c1_ref,
          tab_ref, Wtrig_ref, ba_ref, W0lat_ref, Wf1_ref, Wf2_ref,
          bgam_ref, bbet_ref, W1_ref, b1_ref, W2_ref, b2_ref,
          out_ref, scr):
    def gstep(k, carry):
        base = pl.multiple_of(k * 8, 8)
        for u in range(8):
            r = pl.multiple_of(crow_ref[0, 0, base + u], 8)
            s = sh_ref[0, 0, base + u]
            chunk = tab_ref[pl.ds(r, 8), :]
            rolled = pltpu.roll(chunk, s, axis=0)
            scr[pl.ds(base + u, 1), :] = rolled[u:u + 1, :]
        return carry

    lax.fori_loop(0, B // 8, gstep, 0)

    pcol = pcol_ref[0]
    lane = lax.broadcasted_iota(jnp.int32, (B, 128), 1)
    lidx = (pcol & 31) * 4 + (lane & 3)
    g = jnp.take_along_axis(scr[...], lidx, axis=1)[:, 0:4]

    prow = prow_ref[0]
    xf = (prow & (W - 1)).astype(jnp.float32) * (1.0 / W)
    yf = (prow >> 10).astype(jnp.float32) * (1.0 / H)
    c0 = c0_ref[0, 0]
    c1 = c1_ref[0, 0]

    def six(sa, ca):
        s2 = 2.0 * sa * ca
        c2 = 1.0 - 2.0 * sa * sa
        s3 = sa * c2 + ca * s2
        c3 = ca * c2 - sa * s2
        return s2, c2, s3, c3

    x2 = xf * TWO_PI
    y2 = yf * TWO_PI
    t2 = c0 * TWO_PI
    sx, cx = jnp.sin(x2), jnp.cos(x2)
    sy, cy = jnp.sin(y2), jnp.cos(y2)
    st, ct = jnp.sin(t2), jnp.cos(t2)
    s2x, c2x, s3x, c3x = six(sx, cx)
    s2y, c2y, s3y, c3y = six(sy, cy)
    s2t, c2t, s3t, c3t = six(st, ct)

    zr = jnp.zeros_like(xf)
    stack = jnp.concatenate(
        [xf, yf, sx, cx, s2x, c2x, s3x, c3x,
         sy, cy, s2y, c2y, s3y, c3y,
         c0, c1, st, ct, s2t, c2t, s3t, c3t, zr, zr], axis=0)
    trig_t = jnp.transpose(stack)

    u1 = jnp.dot(trig_t, Wtrig_ref[...],
                 preferred_element_type=jnp.float32) + ba_ref[...]
    h0 = u1[:, 0:64] + jnp.dot(g, W0lat_ref[...],
                               preferred_element_type=jnp.float32)
    film = jnp.maximum(u1[:, 64:112], 0.0)

    gamma = jnp.dot(film, Wf1_ref[...],
                    preferred_element_type=jnp.float32) + bgam_ref[...]
    beta = jnp.dot(film, Wf2_ref[...],
                   preferred_element_type=jnp.float32) + bbet_ref[...]

    def gelu(z):
        return 0.5 * z * (1.0 + lax.erf(z * np.float32(0.7071067811865476)))

    h = gelu(gamma * h0 + beta)
    h2 = gelu(jnp.dot(h, W1_ref[...],
                      preferred_element_type=jnp.float32) + b1_ref[...])
    o = jnp.dot(h2, W2_ref[...],
                preferred_element_type=jnp.float32) + b2_ref[...]
    out_ref[...] = jax.nn.sigmoid(o)


@jax.jit
def kernel(raw_pos, control, latent, Wc, bc, Wt, bt, Wp, bp, Wf, bf,
           W0, b0, W1, b1, W2, b2):
    N = raw_pos.shape[0]
    NB = N // B
    f32 = jnp.float32

    rp3 = raw_pos.astype(jnp.int32).reshape(NB, B, 2)
    p2 = (rp3 * jnp.array([1, W], jnp.int32)).sum(axis=2)
    iota = jnp.arange(B, dtype=jnp.int32).reshape(1, B)
    crow = ((p2 >> 8) << 3).reshape(NB, 1, B)
    sh = (((iota & 7) - ((p2 >> 5) & 7)) & 7).reshape(NB, 1, B)
    pcol = p2.reshape(NB, B, 1)
    prow = p2.reshape(NB, 1, B)
    ctT = control.T.reshape(2, NB, 1, B)
    tab = latent.reshape(H * W * LATENT_C // 128, 128)

    Wtrig = jnp.zeros((24, 112), f32)
    Wtrig = Wtrig.at[0, 0:64].set(W0[4])
    Wtrig = Wtrig.at[1, 0:64].set(W0[5])
    Wtrig = Wtrig.at[2, 0:64].set(W0[6])
    Wtrig = Wtrig.at[8, 0:64].set(W0[7])
    Wtrig = Wtrig.at[4, 0:64].set(W0[8])
    Wtrig = Wtrig.at[10, 0:64].set(W0[9])
    Wtrig = Wtrig.at[14, 64:80].set(Wc[0])
    Wtrig = Wtrig.at[15, 64:80].set(Wc[1])
    Wtrig = Wtrig.at[2:14, 80:96].set(Wp)
    Wtrig = Wtrig.at[16:22, 96:112].set(Wt)
    ba = jnp.concatenate([b0, bc, bp, bt]).reshape(1, 112)
    W0lat = W0[0:4]
    Wf1 = Wf[:, 0:64]
    Wf2 = Wf[:, 64:128]
    bgam = bf[0:64].reshape(1, 64)
    bbet = bf[64:128].reshape(1, 64)
    b1m = b1.reshape(1, 64)
    b2m = b2.reshape(1, 4)

    full = lambda shp: pl.BlockSpec(shp, lambda i: tuple(0 for _ in shp))
    grid_spec = pltpu.PrefetchScalarGridSpec(
        num_scalar_prefetch=0,
        grid=(NB,),
        in_specs=[
            pl.BlockSpec((1, 1, B), lambda i: (i, 0, 0),
                         memory_space=pltpu.SMEM),
            pl.BlockSpec((1, 1, B), lambda i: (i, 0, 0),
                         memory_space=pltpu.SMEM),
            pl.BlockSpec((1, B, 1), lambda i: (i, 0, 0)),
            pl.BlockSpec((1, 1, B), lambda i: (i, 0, 0)),
            pl.BlockSpec((1, 1, 1, B), lambda i: (0, i, 0, 0)),
            pl.BlockSpec((1, 1, 1, B), lambda i: (1, i, 0, 0)),
            full((H * W * LATENT_C // 128, 128)),
            full((24, 112)),
            full((1, 112)),
            full((4, 64)),
            full((48, 64)),
            full((48, 64)),
            full((1, 64)),
            full((1, 64)),
            full((64, 64)),
            full((1, 64)),
            full((64, 4)),
            full((1, 4)),
        ],
        out_specs=pl.BlockSpec((B, 4), lambda i: (i, 0)),
        scratch_shapes=[pltpu.VMEM((B, 128), f32)],
    )
    out = pl.pallas_call(
        _body,
        grid_spec=grid_spec,
        out_shape=jax.ShapeDtypeStruct((N, 4), f32),
        compiler_params=pltpu.CompilerParams(
            dimension_semantics=("parallel",),
        ),
    )(crow, sh, pcol, prow, ctT, ctT, tab, Wtrig, ba, W0lat, Wf1, Wf2,
      bgam, bbet, W1, b1m, W2, b2m)
    return out


# speedup vs baseline: 1.6331x; 1.1332x over previous
import jax
import jax.numpy as jnp
import numpy as np
from jax import lax
from jax.experimental import pallas as pl
from jax.experimental.pallas import tpu as pltpu

H = 1024
W = 1024
LATENT_C = 4
HID = 64
TWO_PI = 2.0 * np.pi
B = 1024


def _body(crow_ref, sh_ref, pcol_ref, prow_ref, c0_ref, c1_ref,
          tab_ref, Wtrig_ref, ba_ref, W0lat_ref, Wf1_ref, Wf2_ref,
          bgam_ref, bbet_ref, W1_ref, b1_ref, W2_ref, b2_ref,
          out_ref, scr):
    def gstep(k, carry):
        base = pl.multiple_of(k * 8, 8)
        for u in range(8):
            r = pl.multiple_of(crow_ref[0, 0, base + u], 8)
            s = sh_ref[0, 0, base + u]
            chunk = tab_ref[pl.ds(r, 8), :]
            rolled = pltpu.roll(chunk, s, axis=0)
            scr[pl.ds(base + u, 1), :] = rolled[u:u + 1, :]
        return carry

    lax.fori_loop(0, B // 8, gstep, 0)

    pcol = pcol_ref[0]
    lane = lax.broadcasted_iota(jnp.int32, (B, 128), 1)
    lidx = (pcol & 31) * 4 + (lane & 3)
    g = jnp.take_along_axis(scr[...], lidx, axis=1)[:, 0:4]

    prow = prow_ref[0]
    xf = (prow & (W - 1)).astype(jnp.float32) * (1.0 / W)
    yf = (prow >> 10).astype(jnp.float32) * (1.0 / H)
    c0 = c0_ref[0, 0]
    c1 = c1_ref[0, 0]

    def six(sa, ca):
        s2 = 2.0 * sa * ca
        c2 = 1.0 - 2.0 * sa * sa
        s3 = sa * c2 + ca * s2
        c3 = ca * c2 - sa * s2
        return s2, c2, s3, c3

    x2 = xf * TWO_PI
    y2 = yf * TWO_PI
    t2 = c0 * TWO_PI
    sx, cx = jnp.sin(x2), jnp.cos(x2)
    sy, cy = jnp.sin(y2), jnp.cos(y2)
    st, ct = jnp.sin(t2), jnp.cos(t2)
    s2x, c2x, s3x, c3x = six(sx, cx)
    s2y, c2y, s3y, c3y = six(sy, cy)
    s2t, c2t, s3t, c3t = six(st, ct)

    zr = jnp.zeros_like(xf)
    stack = jnp.concatenate(
        [xf, yf, sx, cx, s2x, c2x, s3x, c3x,
         sy, cy, s2y, c2y, s3y, c3y,
         c0, c1, st, ct, s2t, c2t, s3t, c3t, zr, zr], axis=0)

    def mm(a, b):
        return lax.dot_general(a, b, (((1,), (0,)), ((), ())),
                               preferred_element_type=jnp.float32)

    u1 = mm(Wtrig_ref[...], stack) + ba_ref[...]
    h0lat = lax.dot_general(W0lat_ref[...], g, (((1,), (1,)), ((), ())),
                            preferred_element_type=jnp.float32)
    h0 = u1[0:64, :] + h0lat
    film = jnp.maximum(u1[64:112, :], 0.0)

    gamma = mm(Wf1_ref[...], film) + bgam_ref[...]
    beta = mm(Wf2_ref[...], film) + bbet_ref[...]

    def gelu(z):
        return 0.5 * z * (1.0 + lax.erf(z * np.float32(0.7071067811865476)))

    h = gelu(gamma * h0 + beta)
    h2 = gelu(mm(W1_ref[...], h) + b1_ref[...])
    o = mm(W2_ref[...], h2) + b2_ref[...]
    out_ref[...] = jax.nn.sigmoid(o)


@jax.jit
def kernel(raw_pos, control, latent, Wc, bc, Wt, bt, Wp, bp, Wf, bf,
           W0, b0, W1, b1, W2, b2):
    N = raw_pos.shape[0]
    NB = N // B
    f32 = jnp.float32

    rp3 = raw_pos.astype(jnp.int32).reshape(NB, B, 2)
    p2 = (rp3 * jnp.array([1, W], jnp.int32)).sum(axis=2)
    iota = jnp.arange(B, dtype=jnp.int32).reshape(1, B)
    crow = ((p2 >> 8) << 3).reshape(NB, 1, B)
    sh = (((iota & 7) - ((p2 >> 5) & 7)) & 7).reshape(NB, 1, B)
    pcol = p2.reshape(NB, B, 1)
    prow = p2.reshape(NB, 1, B)
    ctT = control.T.reshape(2, NB, 1, B)
    tab = latent.reshape(H * W * LATENT_C // 128, 128)

    Wtrig = jnp.zeros((24, 112), f32)
    Wtrig = Wtrig.at[0, 0:64].set(W0[4])
    Wtrig = Wtrig.at[1, 0:64].set(W0[5])
    Wtrig = Wtrig.at[2, 0:64].set(W0[6])
    Wtrig = Wtrig.at[8, 0:64].set(W0[7])
    Wtrig = Wtrig.at[4, 0:64].set(W0[8])
    Wtrig = Wtrig.at[10, 0:64].set(W0[9])
    Wtrig = Wtrig.at[14, 64:80].set(Wc[0])
    Wtrig = Wtrig.at[15, 64:80].set(Wc[1])
    Wtrig = Wtrig.at[2:14, 80:96].set(Wp)
    Wtrig = Wtrig.at[16:22, 96:112].set(Wt)
    WtrigT = Wtrig.T
    ba = jnp.concatenate([b0, bc, bp, bt]).reshape(112, 1)
    W0latT = W0[0:4].T
    Wf1T = Wf[:, 0:64].T
    Wf2T = Wf[:, 64:128].T
    bgam = bf[0:64].reshape(64, 1)
    bbet = bf[64:128].reshape(64, 1)
    W1T = W1.T
    b1m = b1.reshape(64, 1)
    W2T = W2.T
    b2m = b2.reshape(4, 1)

    full = lambda shp: pl.BlockSpec(shp, lambda i: tuple(0 for _ in shp))
    grid_spec = pltpu.PrefetchScalarGridSpec(
        num_scalar_prefetch=0,
        grid=(NB,),
        in_specs=[
            pl.BlockSpec((1, 1, B), lambda i: (i, 0, 0),
                         memory_space=pltpu.SMEM),
            pl.BlockSpec((1, 1, B), lambda i: (i, 0, 0),
                         memory_space=pltpu.SMEM),
            pl.BlockSpec((1, B, 1), lambda i: (i, 0, 0)),
            pl.BlockSpec((1, 1, B), lambda i: (i, 0, 0)),
            pl.BlockSpec((1, 1, 1, B), lambda i: (0, i, 0, 0)),
            pl.BlockSpec((1, 1, 1, B), lambda i: (1, i, 0, 0)),
            full((H * W * LATENT_C // 128, 128)),
            full((112, 24)),
            full((112, 1)),
            full((64, 4)),
            full((64, 48)),
            full((64, 48)),
            full((64, 1)),
            full((64, 1)),
            full((64, 64)),
            full((64, 1)),
            full((4, 64)),
            full((4, 1)),
        ],
        out_specs=pl.BlockSpec((4, B), lambda i: (0, i)),
        scratch_shapes=[pltpu.VMEM((B, 128), f32)],
    )
    out = pl.pallas_call(
        _body,
        grid_spec=grid_spec,
        out_shape=jax.ShapeDtypeStruct((4, N), f32),
        compiler_params=pltpu.CompilerParams(
            dimension_semantics=("parallel",),
        ),
    )(crow, sh, pcol, prow, ctT, ctT, tab, WtrigT, ba, W0latT, Wf1T, Wf2T,
      bgam, bbet, W1T, b1m, W2T, b2m)
    return out.T


# speedup vs baseline: 1.7446x; 1.0683x over previous
import jax
import jax.numpy as jnp
import numpy as np
from jax import lax
from jax.experimental import pallas as pl
from jax.experimental.pallas import tpu as pltpu

H = 1024
W = 1024
LATENT_C = 4
HID = 64
TWO_PI = 2.0 * np.pi
B = 2048


def _body(crow_ref, sh_ref, pcol_ref, prow_ref, c0_ref, c1_ref,
          tab_ref, Wtrig_ref, ba_ref, W0lat_ref, Wf1_ref, Wf2_ref,
          bgam_ref, bbet_ref, W1_ref, b1_ref, W2_ref, b2_ref,
          out_ref, scr):
    def gstep(k, carry):
        base = pl.multiple_of(k * 8, 8)
        for u in range(8):
            r = pl.multiple_of(crow_ref[0, 0, base + u], 8)
            s = sh_ref[0, 0, base + u]
            chunk = tab_ref[pl.ds(r, 8), :]
            rolled = pltpu.roll(chunk, s, axis=0)
            scr[pl.ds(base + u, 1), :] = rolled[u:u + 1, :]
        return carry

    lax.fori_loop(0, B // 8, gstep, 0)

    pcol = pcol_ref[0]
    lane = lax.broadcasted_iota(jnp.int32, (B, 128), 1)
    lidx = (pcol & 31) * 4 + (lane & 3)
    g = jnp.take_along_axis(scr[...], lidx, axis=1)[:, 0:4]

    prow = prow_ref[0]
    xf = (prow & (W - 1)).astype(jnp.float32) * (1.0 / W)
    yf = (prow >> 10).astype(jnp.float32) * (1.0 / H)
    c0 = c0_ref[0, 0]
    c1 = c1_ref[0, 0]

    def six(sa, ca):
        s2 = 2.0 * sa * ca
        c2 = 1.0 - 2.0 * sa * sa
        s3 = sa * c2 + ca * s2
        c3 = ca * c2 - sa * s2
        return s2, c2, s3, c3

    x2 = xf * TWO_PI
    y2 = yf * TWO_PI
    t2 = c0 * TWO_PI
    sx, cx = jnp.sin(x2), jnp.cos(x2)
    sy, cy = jnp.sin(y2), jnp.cos(y2)
    st, ct = jnp.sin(t2), jnp.cos(t2)
    s2x, c2x, s3x, c3x = six(sx, cx)
    s2y, c2y, s3y, c3y = six(sy, cy)
    s2t, c2t, s3t, c3t = six(st, ct)

    zr = jnp.zeros_like(xf)
    stack = jnp.concatenate(
        [xf, yf, sx, cx, s2x, c2x, s3x, c3x,
         sy, cy, s2y, c2y, s3y, c3y,
         c0, c1, st, ct, s2t, c2t, s3t, c3t, zr, zr], axis=0)

    def mm(a, b):
        return lax.dot_general(a, b, (((1,), (0,)), ((), ())),
                               preferred_element_type=jnp.float32)

    u1 = mm(Wtrig_ref[...], stack) + ba_ref[...]
    h0lat = lax.dot_general(W0lat_ref[...], g, (((1,), (1,)), ((), ())),
                            preferred_element_type=jnp.float32)
    h0 = u1[0:64, :] + h0lat
    film = jnp.maximum(u1[64:112, :], 0.0)

    gamma = mm(Wf1_ref[...], film) + bgam_ref[...]
    beta = mm(Wf2_ref[...], film) + bbet_ref[...]

    def gelu(z):
        return 0.5 * z * (1.0 + lax.erf(z * np.float32(0.7071067811865476)))

    h = gelu(gamma * h0 + beta)
    h2 = gelu(mm(W1_ref[...], h) + b1_ref[...])
    o = mm(W2_ref[...], h2) + b2_ref[...]
    out_ref[...] = jax.nn.sigmoid(o)


@jax.jit
def kernel(raw_pos, control, latent, Wc, bc, Wt, bt, Wp, bp, Wf, bf,
           W0, b0, W1, b1, W2, b2):
    N = raw_pos.shape[0]
    NB = N // B
    f32 = jnp.float32

    rp3 = raw_pos.astype(jnp.int32).reshape(NB, B, 2)
    p2 = (rp3 * jnp.array([1, W], jnp.int32)).sum(axis=2)
    iota = jnp.arange(B, dtype=jnp.int32).reshape(1, B)
    crow = ((p2 >> 8) << 3).reshape(NB, 1, B)
    sh = (((iota & 7) - ((p2 >> 5) & 7)) & 7).reshape(NB, 1, B)
    pcol = p2.reshape(NB, B, 1)
    prow = p2.reshape(NB, 1, B)
    ctT = control.T.reshape(2, NB, 1, B)
    tab = latent.reshape(H * W * LATENT_C // 128, 128)

    Wtrig = jnp.zeros((24, 112), f32)
    Wtrig = Wtrig.at[0, 0:64].set(W0[4])
    Wtrig = Wtrig.at[1, 0:64].set(W0[5])
    Wtrig = Wtrig.at[2, 0:64].set(W0[6])
    Wtrig = Wtrig.at[8, 0:64].set(W0[7])
    Wtrig = Wtrig.at[4, 0:64].set(W0[8])
    Wtrig = Wtrig.at[10, 0:64].set(W0[9])
    Wtrig = Wtrig.at[14, 64:80].set(Wc[0])
    Wtrig = Wtrig.at[15, 64:80].set(Wc[1])
    Wtrig = Wtrig.at[2:14, 80:96].set(Wp)
    Wtrig = Wtrig.at[16:22, 96:112].set(Wt)
    WtrigT = Wtrig.T
    ba = jnp.concatenate([b0, bc, bp, bt]).reshape(112, 1)
    W0latT = W0[0:4].T
    Wf1T = Wf[:, 0:64].T
    Wf2T = Wf[:, 64:128].T
    bgam = bf[0:64].reshape(64, 1)
    bbet = bf[64:128].reshape(64, 1)
    W1T = W1.T
    b1m = b1.reshape(64, 1)
    W2T = W2.T
    b2m = b2.reshape(4, 1)

    full = lambda shp: pl.BlockSpec(shp, lambda i: tuple(0 for _ in shp))
    grid_spec = pltpu.PrefetchScalarGridSpec(
        num_scalar_prefetch=0,
        grid=(NB,),
        in_specs=[
            pl.BlockSpec((1, 1, B), lambda i: (i, 0, 0),
                         memory_space=pltpu.SMEM),
            pl.BlockSpec((1, 1, B), lambda i: (i, 0, 0),
                         memory_space=pltpu.SMEM),
            pl.BlockSpec((1, B, 1), lambda i: (i, 0, 0)),
            pl.BlockSpec((1, 1, B), lambda i: (i, 0, 0)),
            pl.BlockSpec((1, 1, 1, B), lambda i: (0, i, 0, 0)),
            pl.BlockSpec((1, 1, 1, B), lambda i: (1, i, 0, 0)),
            full((H * W * LATENT_C // 128, 128)),
            full((112, 24)),
            full((112, 1)),
            full((64, 4)),
            full((64, 48)),
            full((64, 48)),
            full((64, 1)),
            full((64, 1)),
            full((64, 64)),
            full((64, 1)),
            full((4, 64)),
            full((4, 1)),
        ],
        out_specs=pl.BlockSpec((4, B), lambda i: (0, i)),
        scratch_shapes=[pltpu.VMEM((B, 128), f32)],
    )
    out = pl.pallas_call(
        _body,
        grid_spec=grid_spec,
        out_shape=jax.ShapeDtypeStruct((4, N), f32),
        compiler_params=pltpu.CompilerParams(
            dimension_semantics=("parallel",),
        ),
    )(crow, sh, pcol, prow, ctT, ctT, tab, WtrigT, ba, W0latT, Wf1T, Wf2T,
      bgam, bbet, W1T, b1m, W2T, b2m)
    return out.T


# speedup vs baseline: 1.7989x; 1.0311x over previous
import jax
import jax.numpy as jnp
import numpy as np
from jax import lax
from jax.experimental import pallas as pl
from jax.experimental.pallas import tpu as pltpu

H = 1024
W = 1024
LATENT_C = 4
HID = 64
TWO_PI = 2.0 * np.pi
B = 4096


def _body(crow_ref, sh_ref, pcol_ref, prow_ref, c0_ref, c1_ref,
          tab_ref, Wtrig_ref, ba_ref, W0lat_ref, Wf1_ref, Wf2_ref,
          bgam_ref, bbet_ref, W1_ref, b1_ref, W2_ref, b2_ref,
          out_ref, scr):
    def gstep(k, carry):
        base = pl.multiple_of(k * 8, 8)
        for u in range(8):
            r = pl.multiple_of(crow_ref[0, 0, base + u], 8)
            s = sh_ref[0, 0, base + u]
            chunk = tab_ref[pl.ds(r, 8), :]
            rolled = pltpu.roll(chunk, s, axis=0)
            scr[pl.ds(base + u, 1), :] = rolled[u:u + 1, :]
        return carry

    lax.fori_loop(0, B // 8, gstep, 0)

    pcol = pcol_ref[0]
    lane = lax.broadcasted_iota(jnp.int32, (B, 128), 1)
    lidx = (pcol & 31) * 4 + (lane & 3)
    g = jnp.take_along_axis(scr[...], lidx, axis=1)[:, 0:4]

    prow = prow_ref[0]
    xf = (prow & (W - 1)).astype(jnp.float32) * (1.0 / W)
    yf = (prow >> 10).astype(jnp.float32) * (1.0 / H)
    c0 = c0_ref[0, 0]
    c1 = c1_ref[0, 0]

    def six(sa, ca):
        s2 = 2.0 * sa * ca
        c2 = 1.0 - 2.0 * sa * sa
        s3 = sa * c2 + ca * s2
        c3 = ca * c2 - sa * s2
        return s2, c2, s3, c3

    x2 = xf * TWO_PI
    y2 = yf * TWO_PI
    t2 = c0 * TWO_PI
    sx, cx = jnp.sin(x2), jnp.cos(x2)
    sy, cy = jnp.sin(y2), jnp.cos(y2)
    st, ct = jnp.sin(t2), jnp.cos(t2)
    s2x, c2x, s3x, c3x = six(sx, cx)
    s2y, c2y, s3y, c3y = six(sy, cy)
    s2t, c2t, s3t, c3t = six(st, ct)

    zr = jnp.zeros_like(xf)
    stack = jnp.concatenate(
        [xf, yf, sx, cx, s2x, c2x, s3x, c3x,
         sy, cy, s2y, c2y, s3y, c3y,
         c0, c1, st, ct, s2t, c2t, s3t, c3t, zr, zr], axis=0)

    def mm(a, b):
        return lax.dot_general(a, b, (((1,), (0,)), ((), ())),
                               preferred_element_type=jnp.float32)

    u1 = mm(Wtrig_ref[...], stack) + ba_ref[...]
    h0lat = lax.dot_general(W0lat_ref[...], g, (((1,), (1,)), ((), ())),
                            preferred_element_type=jnp.float32)
    h0 = u1[0:64, :] + h0lat
    film = jnp.maximum(u1[64:112, :], 0.0)

    gamma = mm(Wf1_ref[...], film) + bgam_ref[...]
    beta = mm(Wf2_ref[...], film) + bbet_ref[...]

    def gelu(z):
        return 0.5 * z * (1.0 + lax.erf(z * np.float32(0.7071067811865476)))

    h = gelu(gamma * h0 + beta)
    h2 = gelu(mm(W1_ref[...], h) + b1_ref[...])
    o = mm(W2_ref[...], h2) + b2_ref[...]
    out_ref[...] = jax.nn.sigmoid(o)


@jax.jit
def kernel(raw_pos, control, latent, Wc, bc, Wt, bt, Wp, bp, Wf, bf,
           W0, b0, W1, b1, W2, b2):
    N = raw_pos.shape[0]
    NB = N // B
    f32 = jnp.float32

    rp3 = raw_pos.astype(jnp.int32).reshape(NB, B, 2)
    p2 = (rp3 * jnp.array([1, W], jnp.int32)).sum(axis=2)
    iota = jnp.arange(B, dtype=jnp.int32).reshape(1, B)
    crow = ((p2 >> 8) << 3).reshape(NB, 1, B)
    sh = (((iota & 7) - ((p2 >> 5) & 7)) & 7).reshape(NB, 1, B)
    pcol = p2.reshape(NB, B, 1)
    prow = p2.reshape(NB, 1, B)
    ctT = control.T.reshape(2, NB, 1, B)
    tab = latent.reshape(H * W * LATENT_C // 128, 128)

    Wtrig = jnp.zeros((24, 112), f32)
    Wtrig = Wtrig.at[0, 0:64].set(W0[4])
    Wtrig = Wtrig.at[1, 0:64].set(W0[5])
    Wtrig = Wtrig.at[2, 0:64].set(W0[6])
    Wtrig = Wtrig.at[8, 0:64].set(W0[7])
    Wtrig = Wtrig.at[4, 0:64].set(W0[8])
    Wtrig = Wtrig.at[10, 0:64].set(W0[9])
    Wtrig = Wtrig.at[14, 64:80].set(Wc[0])
    Wtrig = Wtrig.at[15, 64:80].set(Wc[1])
    Wtrig = Wtrig.at[2:14, 80:96].set(Wp)
    Wtrig = Wtrig.at[16:22, 96:112].set(Wt)
    WtrigT = Wtrig.T
    ba = jnp.concatenate([b0, bc, bp, bt]).reshape(112, 1)
    W0latT = W0[0:4].T
    Wf1T = Wf[:, 0:64].T
    Wf2T = Wf[:, 64:128].T
    bgam = bf[0:64].reshape(64, 1)
    bbet = bf[64:128].reshape(64, 1)
    W1T = W1.T
    b1m = b1.reshape(64, 1)
    W2T = W2.T
    b2m = b2.reshape(4, 1)

    full = lambda shp: pl.BlockSpec(shp, lambda i: tuple(0 for _ in shp))
    grid_spec = pltpu.PrefetchScalarGridSpec(
        num_scalar_prefetch=0,
        grid=(NB,),
        in_specs=[
            pl.BlockSpec((1, 1, B), lambda i: (i, 0, 0),
                         memory_space=pltpu.SMEM),
            pl.BlockSpec((1, 1, B), lambda i: (i, 0, 0),
                         memory_space=pltpu.SMEM),
            pl.BlockSpec((1, B, 1), lambda i: (i, 0, 0)),
            pl.BlockSpec((1, 1, B), lambda i: (i, 0, 0)),
            pl.BlockSpec((1, 1, 1, B), lambda i: (0, i, 0, 0)),
            pl.BlockSpec((1, 1, 1, B), lambda i: (1, i, 0, 0)),
            full((H * W * LATENT_C // 128, 128)),
            full((112, 24)),
            full((112, 1)),
            full((64, 4)),
            full((64, 48)),
            full((64, 48)),
            full((64, 1)),
            full((64, 1)),
            full((64, 64)),
            full((64, 1)),
            full((4, 64)),
            full((4, 1)),
        ],
        out_specs=pl.BlockSpec((4, B), lambda i: (0, i)),
        scratch_shapes=[pltpu.VMEM((B, 128), f32)],
    )
    out = pl.pallas_call(
        _body,
        grid_spec=grid_spec,
        out_shape=jax.ShapeDtypeStruct((4, N), f32),
        compiler_params=pltpu.CompilerParams(
            dimension_semantics=("parallel",),
        ),
    )(crow, sh, pcol, prow, ctT, ctT, tab, WtrigT, ba, W0latT, Wf1T, Wf2T,
      bgam, bbet, W1T, b1m, W2T, b2m)
    return out.T
